# Initial kernel scaffold; baseline (speedup 1.0000x reference)
import jax
import jax.numpy as jnp
from jax.experimental import pallas as pl


def kernel(node_pos, edge_index, apsp, batch_index):
    raise NotImplementedError("write your pallas kernel here")


# speedup vs baseline: 57.5720x; 57.5720x over previous
import functools

import jax
import jax.numpy as jnp
from jax import lax
from jax.experimental import pallas as pl
from jax.experimental.pallas import tpu as pltpu
from jax.experimental.pallas import tpu_sc as plsc

NUM_GRAPHS = 128
LANES = 16
CHUNK = 2000


def _rsqrt_newton(t):
    bits = plsc.bitcast(t, jnp.int32)
    r = plsc.bitcast(jnp.int32(0x5F3759DF) - (bits >> 1), jnp.float32)
    half_t = 0.5 * t
    for _ in range(3):
        r = r * (1.5 - half_t * r * r)
    return r


def _make_sc_call(n_nodes, n_edges, num_workers):
    edges_per_worker = n_edges // num_workers
    n_chunks = edges_per_worker // CHUNK
    groups = CHUNK // LANES

    mesh = plsc.VectorSubcoreMesh(core_axis_name="c", subcore_axis_name="s")

    @functools.partial(
        pl.kernel,
        out_type=jax.ShapeDtypeStruct((num_workers, LANES), jnp.float32),
        mesh=mesh,
        scratch_types=[
            pltpu.VMEM((n_nodes,), jnp.float32),
            pltpu.VMEM((n_nodes,), jnp.float32),
            pltpu.VMEM((CHUNK,), jnp.int32),
            pltpu.VMEM((CHUNK,), jnp.int32),
            pltpu.VMEM((CHUNK,), jnp.float32),
            pltpu.VMEM((LANES,), jnp.float32),
        ],
        compiler_params=pltpu.CompilerParams(needs_layout_passes=False),
    )
    def sc_call(xs_hbm, ys_hbm, e0_hbm, e1_hbm, apsp_hbm, out_hbm,
                xs_v, ys_v, e0_v, e1_v, a_v, acc_v):
        num_cores = lax.axis_size("c")
        wid = lax.axis_index("s") * num_cores + lax.axis_index("c")
        pltpu.sync_copy(xs_hbm, xs_v)
        pltpu.sync_copy(ys_hbm, ys_v)
        base = wid * edges_per_worker

        def chunk_body(ci, acc):
            off = base + ci * CHUNK
            pltpu.sync_copy(e0_hbm.at[pl.ds(off, CHUNK)], e0_v)
            pltpu.sync_copy(e1_hbm.at[pl.ds(off, CHUNK)], e1_v)
            pltpu.sync_copy(apsp_hbm.at[pl.ds(off, CHUNK)], a_v)

            def group_body(g, acc):
                s = pl.ds(g * LANES, LANES)
                i0 = e0_v[s]
                i1 = e1_v[s]
                a = a_v[s]
                sx = plsc.load_gather(xs_v, [i0])
                sy = plsc.load_gather(ys_v, [i0])
                ex = plsc.load_gather(xs_v, [i1])
                ey = plsc.load_gather(ys_v, [i1])
                dx = ex - sx
                dy = ey - sy
                t = dx * dx + dy * dy
                dist = t * _rsqrt_newton(t)
                q = (dist - a) / a
                return acc + q * q

            return lax.fori_loop(0, groups, group_body, acc)

        acc = lax.fori_loop(0, n_chunks, chunk_body,
                            jnp.zeros((LANES,), jnp.float32))
        acc_v[...] = acc
        pltpu.sync_copy(acc_v, out_hbm.at[wid])

    return sc_call


def kernel(node_pos, edge_index, apsp, batch_index):
    del batch_index
    n_nodes = node_pos.shape[0]
    n_edges = apsp.shape[0]
    info = plsc.get_sparse_core_info()
    num_workers = info.num_cores * info.num_subcores
    assert n_edges % (num_workers * CHUNK) == 0
    xs = node_pos[:, 0]
    ys = node_pos[:, 1]
    sc_call = _make_sc_call(n_nodes, n_edges, num_workers)
    partials = sc_call(xs, ys, edge_index[0], edge_index[1], apsp)
    return jnp.sum(partials) / NUM_GRAPHS


# speedup vs baseline: 116.0866x; 2.0164x over previous
import functools

import jax
import jax.numpy as jnp
from jax import lax
from jax.experimental import pallas as pl
from jax.experimental.pallas import tpu as pltpu
from jax.experimental.pallas import tpu_sc as plsc

NUM_GRAPHS = 128
LANES = 16
CHUNK = 2000


def _rsqrt_newton(t):
    bits = plsc.bitcast(t, jnp.int32)
    r = plsc.bitcast(jnp.int32(0x5F3759DF) - (bits >> 1), jnp.float32)
    half_t = 0.5 * t
    for _ in range(3):
        r = r * (1.5 - half_t * r * r)
    return r


def _make_sc_call(n_nodes, n_edges, num_workers):
    edges_per_worker = n_edges // num_workers
    n_chunks = edges_per_worker // CHUNK
    groups = CHUNK // LANES
    assert n_chunks % 2 == 1 and n_chunks >= 3

    mesh = plsc.VectorSubcoreMesh(core_axis_name="c", subcore_axis_name="s")

    @functools.partial(
        pl.kernel,
        out_type=jax.ShapeDtypeStruct((num_workers, LANES), jnp.float32),
        mesh=mesh,
        scratch_types=[
            pltpu.VMEM((n_nodes,), jnp.float32),
            pltpu.VMEM((n_nodes,), jnp.float32),
            pltpu.VMEM((CHUNK,), jnp.int32),
            pltpu.VMEM((CHUNK,), jnp.int32),
            pltpu.VMEM((CHUNK,), jnp.float32),
            pltpu.VMEM((CHUNK,), jnp.int32),
            pltpu.VMEM((CHUNK,), jnp.int32),
            pltpu.VMEM((CHUNK,), jnp.float32),
            pltpu.VMEM((LANES,), jnp.float32),
            pltpu.SemaphoreType.DMA,
            pltpu.SemaphoreType.DMA,
            pltpu.SemaphoreType.DMA,
        ],
        compiler_params=pltpu.CompilerParams(needs_layout_passes=False),
    )
    def sc_call(xs_hbm, ys_hbm, eidx_hbm, apsp_hbm, out_hbm,
                xs_v, ys_v, e0a, e1a, aa, e0b, e1b, ab, acc_v,
                sem_t, sem0, sem1):
        num_cores = lax.axis_size("c")
        wid = lax.axis_index("s") * num_cores + lax.axis_index("c")
        base = wid * edges_per_worker
        buf0 = (e0a, e1a, aa)
        buf1 = (e0b, e1b, ab)

        def issue(ci, bufs, sem):
            off = base + ci * CHUNK
            pltpu.async_copy(eidx_hbm.at[pl.ds(off, CHUNK)], bufs[0], sem)
            pltpu.async_copy(
                eidx_hbm.at[pl.ds(n_edges + off, CHUNK)], bufs[1], sem)
            pltpu.async_copy(apsp_hbm.at[pl.ds(off, CHUNK)], bufs[2], sem)

        def drain(bufs, sem):
            pltpu.make_async_copy(
                eidx_hbm.at[pl.ds(0, CHUNK)], bufs[0], sem).wait()
            pltpu.make_async_copy(
                eidx_hbm.at[pl.ds(0, CHUNK)], bufs[1], sem).wait()
            pltpu.make_async_copy(
                apsp_hbm.at[pl.ds(0, CHUNK)], bufs[2], sem).wait()

        def compute(bufs, acc):
            e0_v, e1_v, a_v = bufs

            def group_body(g, acc):
                s = pl.ds(g * LANES, LANES)
                i0 = e0_v[s]
                i1 = e1_v[s]
                a = a_v[s]
                sx = plsc.load_gather(xs_v, [i0])
                sy = plsc.load_gather(ys_v, [i0])
                ex = plsc.load_gather(xs_v, [i1])
                ey = plsc.load_gather(ys_v, [i1])
                dx = ex - sx
                dy = ey - sy
                t = dx * dx + dy * dy
                dist = t * _rsqrt_newton(t)
                q = (dist - a) / a
                return acc + q * q

            return lax.fori_loop(0, groups, group_body, acc)

        dt0 = pltpu.async_copy(xs_hbm, xs_v, sem_t)
        dt1 = pltpu.async_copy(ys_hbm, ys_v, sem_t)
        issue(0, buf0, sem0)
        issue(1, buf1, sem1)
        dt0.wait()
        dt1.wait()

        def pair_body(i, acc):
            ci = 2 * i
            drain(buf0, sem0)
            acc = compute(buf0, acc)
            issue(ci + 2, buf0, sem0)
            drain(buf1, sem1)
            acc = compute(buf1, acc)

            @pl.when(i < (n_chunks - 1) // 2 - 1)
            def _():
                issue(ci + 3, buf1, sem1)

            return acc

        acc = lax.fori_loop(0, (n_chunks - 1) // 2, pair_body,
                            jnp.zeros((LANES,), jnp.float32))
        drain(buf0, sem0)
        acc = compute(buf0, acc)
        acc_v[...] = acc
        pltpu.sync_copy(acc_v, out_hbm.at[wid])

    return sc_call


def kernel(node_pos, edge_index, apsp, batch_index):
    del batch_index
    n_nodes = node_pos.shape[0]
    n_edges = apsp.shape[0]
    info = plsc.get_sparse_core_info()
    num_workers = info.num_cores * info.num_subcores
    assert n_edges % (num_workers * CHUNK) == 0
    xs = node_pos[:, 0]
    ys = node_pos[:, 1]
    sc_call = _make_sc_call(n_nodes, n_edges, num_workers)
    partials = sc_call(xs, ys, edge_index.reshape(-1), apsp)
    return jnp.sum(partials) / NUM_GRAPHS


# speedup vs baseline: 173.7590x; 1.4968x over previous
import functools

import jax
import jax.numpy as jnp
from jax import lax
from jax.experimental import pallas as pl
from jax.experimental.pallas import tpu as pltpu
from jax.experimental.pallas import tpu_sc as plsc

NUM_GRAPHS = 128
LANES = 16
CHUNK = 2048
NBUF = 4


def _rsqrt_newton(t):
    bits = plsc.bitcast(t, jnp.int32)
    r = plsc.bitcast(jnp.int32(0x5F3759DF) - (bits >> 1), jnp.float32)
    half_t = 0.5 * t
    for _ in range(3):
        r = r * (1.5 - half_t * r * r)
    return r


def _edge_stress(i0, i1, a, xs_v, ys_v):
    sx = plsc.load_gather(xs_v, [i0])
    sy = plsc.load_gather(ys_v, [i0])
    ex = plsc.load_gather(xs_v, [i1])
    ey = plsc.load_gather(ys_v, [i1])
    dx = ex - sx
    dy = ey - sy
    t = dx * dx + dy * dy
    dist = t * _rsqrt_newton(t)
    q = (dist - a) / a
    return q * q


def _make_sc_call(n_nodes, n_edges, num_workers):
    n_full = n_edges // CHUNK
    n_ring = n_full // num_workers
    n_extra = n_full - n_ring * num_workers
    rem = n_edges - n_full * CHUNK
    groups = CHUNK // LANES
    assert n_ring % NBUF == 0 and n_ring > NBUF
    assert 0 < n_extra < num_workers
    assert rem == num_workers * LANES

    mesh = plsc.VectorSubcoreMesh(core_axis_name="c", subcore_axis_name="s")

    @functools.partial(
        pl.kernel,
        out_type=jax.ShapeDtypeStruct((num_workers, LANES), jnp.float32),
        mesh=mesh,
        scratch_types=[
            pltpu.VMEM((n_nodes,), jnp.float32),
            pltpu.VMEM((n_nodes,), jnp.float32),
            [pltpu.VMEM((2, CHUNK), jnp.int32) for _ in range(NBUF)],
            [pltpu.VMEM((CHUNK,), jnp.float32) for _ in range(NBUF)],
            pltpu.VMEM((LANES,), jnp.float32),
            pltpu.SemaphoreType.DMA,
            [pltpu.SemaphoreType.DMA for _ in range(NBUF)],
        ],
        compiler_params=pltpu.CompilerParams(needs_layout_passes=False),
    )
    def sc_call(xs_hbm, ys_hbm, eidx_hbm, apsp_hbm, out_hbm,
                xs_v, ys_v, ebufs, abufs, acc_v, sem_t, sems):
        num_cores = lax.axis_size("c")
        wid = lax.axis_index("s") * num_cores + lax.axis_index("c")

        def issue(ci, b, size=CHUNK):
            off = ci * CHUNK
            pltpu.async_copy(
                eidx_hbm.at[:, pl.ds(off, size)],
                ebufs[b].at[:, pl.ds(0, size)], sems[b])
            pltpu.async_copy(
                apsp_hbm.at[pl.ds(off, size)],
                abufs[b].at[pl.ds(0, size)], sems[b])

        def drain(b, size=CHUNK):
            pltpu.make_async_copy(
                eidx_hbm.at[:, pl.ds(0, size)],
                ebufs[b].at[:, pl.ds(0, size)], sems[b]).wait()
            pltpu.make_async_copy(
                apsp_hbm.at[pl.ds(0, size)],
                abufs[b].at[pl.ds(0, size)], sems[b]).wait()

        def compute(b, acc):
            e_v = ebufs[b]
            a_v = abufs[b]

            def group_body(g, acc):
                s = pl.ds(g * LANES, LANES)
                return acc + _edge_stress(
                    e_v[0, s], e_v[1, s], a_v[s], xs_v, ys_v)

            return lax.fori_loop(0, groups, group_body, acc)

        dt0 = pltpu.async_copy(xs_hbm, xs_v, sem_t)
        dt1 = pltpu.async_copy(ys_hbm, ys_v, sem_t)
        for b in range(NBUF):
            issue(wid + num_workers * b, b)
        dt0.wait()
        dt1.wait()

        last_round = n_ring // NBUF - 1
        extra_ci = n_ring * num_workers + jnp.minimum(wid, n_extra - 1)

        def round_body(rnd, acc):
            for b in range(NBUF):
                drain(b)
                acc = compute(b, acc)

                @pl.when(rnd < last_round)
                def _():
                    issue(wid + num_workers * (NBUF * rnd + b + NBUF), b)

                if b == 0:
                    @pl.when(rnd == last_round)
                    def _():
                        issue(extra_ci, 0)
                elif b == 1:
                    @pl.when(rnd == last_round)
                    def _():
                        issue(n_full, 1, size=rem)

            return acc

        acc = lax.fori_loop(0, last_round + 1, round_body,
                            jnp.zeros((LANES,), jnp.float32))
        drain(0)
        extra = compute(0, jnp.zeros((LANES,), jnp.float32))
        acc = acc + jnp.where(
            jnp.broadcast_to(wid < n_extra, (LANES,)), extra, 0.0)
        drain(1, size=rem)
        s = pl.ds(wid * LANES, LANES)
        acc = acc + _edge_stress(
            ebufs[1][0, s], ebufs[1][1, s], abufs[1][s], xs_v, ys_v)
        acc_v[...] = acc
        pltpu.sync_copy(acc_v, out_hbm.at[wid])

    return sc_call


def kernel(node_pos, edge_index, apsp, batch_index):
    del batch_index
    n_nodes = node_pos.shape[0]
    n_edges = apsp.shape[0]
    info = plsc.get_sparse_core_info()
    num_workers = info.num_cores * info.num_subcores
    xs = node_pos[:, 0]
    ys = node_pos[:, 1]
    sc_call = _make_sc_call(n_nodes, n_edges, num_workers)
    partials = sc_call(xs, ys, edge_index, apsp)
    return jnp.sum(partials) / NUM_GRAPHS
